# baseline (speedup 1.0000x reference)
import functools

import jax
import jax.numpy as jnp
from jax import lax
from jax.experimental import pallas as pl
from jax.experimental.pallas import tpu as pltpu
from jax.experimental.pallas import tpu_sc as plsc

L = 16
ROWS_PER_TILE = 512
IDX_PER_BATCH = 512
CHUNK = 16
NCH = IDX_PER_BATCH // CHUNK
NSUB = 32
SUB = ROWS_PER_TILE // NSUB


def _sc_body(img_hbm, text_hbm, vi_hbm, out_hbm,
             idx_v, winner, list_t, list_j, buf_a, buf_b, buf_c,
             sem_s, semg0, semg1, semg2, semp0, semp1, semp2):
    nc = 2
    wid = lax.axis_index("s") * nc + lax.axis_index("c")
    b = wid // 8
    seg = wid % 8
    lo = wid * ROWS_PER_TILE
    seg_lo = seg * ROWS_PER_TILE

    pltpu.sync_copy(vi_hbm.at[pl.ds(b * IDX_PER_BATCH, IDX_PER_BATCH)], idx_v)

    iota = lax.iota(jnp.int32, L)

    for c in range(IDX_PER_BATCH // L):
        v = idx_v[pl.ds(c * L, L)]
        jl = iota + c * L
        valid = ((v >= seg_lo) & (v < seg_lo + ROWS_PER_TILE) & (v != 0))
        addr = jnp.clip(v - seg_lo, 0, ROWS_PER_TILE - 1)
        for lane in range(L):
            plsc.store_scatter(winner, [addr], jl,
                               mask=valid & (iota == lane))

    cnt = jnp.int32(0)
    for c in range(IDX_PER_BATCH // L):
        v = idx_v[pl.ds(c * L, L)]
        jl = iota + c * L
        valid0 = (v >= seg_lo) & (v < seg_lo + ROWS_PER_TILE) & (v != 0)
        addr = jnp.clip(v - seg_lo, 0, ROWS_PER_TILE - 1)
        w = plsc.load_gather(winner, [addr], mask=valid0)
        keep = valid0 & (w == jl)
        mi = keep.astype(jnp.int32)
        incl = plsc.cumsum(mi)
        pos = cnt + incl - mi
        plsc.store_scatter(list_t, [pos // CHUNK, pos % CHUNK],
                           b * 4096 + v, mask=keep)
        plsc.store_scatter(list_j, [pos // CHUNK, pos % CHUNK],
                           b * IDX_PER_BATCH + jl, mask=keep)
        cnt = cnt + jnp.sum(mi)

    n = cnt
    ceil = ((n + CHUNK - 1) // CHUNK) * CHUNK
    last_i = jnp.maximum(n - 1, 0)
    lt = plsc.load_gather(
        list_t, [jnp.full((L,), last_i // CHUNK, jnp.int32),
                 jnp.full((L,), last_i % CHUNK, jnp.int32)])
    lj = plsc.load_gather(
        list_j, [jnp.full((L,), last_i // CHUNK, jnp.int32),
                 jnp.full((L,), last_i % CHUNK, jnp.int32)])
    for k in range(CHUNK // L):
        pos2 = n + k * L + iota
        m = pos2 < ceil
        pc = jnp.clip(pos2, 0, IDX_PER_BATCH - 1)
        plsc.store_scatter(list_t, [pc // CHUNK, pc % CHUNK], lt, mask=m)
        plsc.store_scatter(list_j, [pc // CHUNK, pc % CHUNK], lj, mask=m)

    NBUF = 3
    bufs = (buf_a, buf_b, buf_c)
    semg = (semg0, semg1, semg2)
    semp = (semp0, semp1, semp2)
    g = [None] * NSUB
    p = [None] * NSUB
    for k in range(min(NBUF - 1, NSUB)):
        g[k] = pltpu.async_copy(
            text_hbm.at[pl.ds(lo + k * SUB, SUB)], bufs[k], semg[k])
    for k in range(NSUB):
        x = k % NBUF
        ahead = k + NBUF - 1
        if ahead < NSUB:
            if ahead >= NBUF:
                p[ahead - NBUF].wait()
            g[ahead] = pltpu.async_copy(
                text_hbm.at[pl.ds(lo + ahead * SUB, SUB)],
                bufs[ahead % NBUF], semg[ahead % NBUF])
        g[k].wait()
        p[k] = pltpu.async_copy(
            bufs[x], out_hbm.at[pl.ds(lo + k * SUB, SUB)], semp[x])
    for k in range(max(0, NSUB - NBUF), NSUB):
        p[k].wait()

    nch = ceil // CHUNK

    def chunk_body(k, carry):
        pltpu.async_copy(img_hbm.at[list_j.at[k]], buf_a, semg0).wait()
        pltpu.async_copy(buf_a, out_hbm.at[list_t.at[k]], semp0).wait()
        return carry

    lax.fori_loop(0, nch, chunk_body, jnp.int32(0))


@jax.jit
def _interleave(img_flat, text_flat, vi_flat):
    mesh = plsc.VectorSubcoreMesh(core_axis_name="c", subcore_axis_name="s")
    kern = pl.kernel(
        _sc_body,
        out_type=jax.ShapeDtypeStruct(text_flat.shape, text_flat.dtype),
        mesh=mesh,
        scratch_types=[
            pltpu.VMEM((IDX_PER_BATCH,), jnp.int32),
            pltpu.VMEM((ROWS_PER_TILE,), jnp.int32),
            pltpu.VMEM((NCH, CHUNK), jnp.int32),
            pltpu.VMEM((NCH, CHUNK), jnp.int32),
            pltpu.VMEM((SUB, 2048), jnp.float32),
            pltpu.VMEM((SUB, 2048), jnp.float32),
            pltpu.VMEM((SUB, 2048), jnp.float32),
            pltpu.SemaphoreType.DMA,
            pltpu.SemaphoreType.DMA,
            pltpu.SemaphoreType.DMA,
            pltpu.SemaphoreType.DMA,
            pltpu.SemaphoreType.DMA,
            pltpu.SemaphoreType.DMA,
            pltpu.SemaphoreType.DMA,
        ],
        compiler_params=pltpu.CompilerParams(needs_layout_passes=False),
    )
    return kern(img_flat, text_flat, vi_flat)


def kernel(image_embeddings, text_embeddings, vision_indices):
    B, S, D = text_embeddings.shape
    img_flat = image_embeddings.reshape(-1, D)
    text_flat = text_embeddings.reshape(B * S, D)
    vi_flat = vision_indices.astype(jnp.int32).reshape(-1)
    out = _interleave(img_flat, text_flat, vi_flat)
    return out.reshape(B, S, D)


# speedup vs baseline: 1.0815x; 1.0815x over previous
import jax
import jax.numpy as jnp
from jax import lax
from jax.experimental import pallas as pl
from jax.experimental.pallas import tpu as pltpu
from jax.experimental.pallas import tpu_sc as plsc
from jax._src.pallas import mpmd as pl_mpmd

L = 16
ROWS_PER_TILE = 512
IDX_PER_BATCH = 512
CHUNK = 16
NCH = IDX_PER_BATCH // CHUNK


def _copy_body(t_ref, o_ref):
    o_ref[...] = t_ref[...]


def _sc_body(img_hbm, copied_hbm, vi_hbm, out_hbm,
             idx_v, winner, list_t, list_j, buf_a, buf_b,
             semg0, semg1, semp0, semp1):
    del copied_hbm
    nc = 2
    wid = lax.axis_index("s") * nc + lax.axis_index("c")
    b = wid // 8
    seg = wid % 8
    seg_lo = seg * ROWS_PER_TILE

    pltpu.sync_copy(vi_hbm.at[pl.ds(b * IDX_PER_BATCH, IDX_PER_BATCH)], idx_v)

    iota = lax.iota(jnp.int32, L)

    for c in range(IDX_PER_BATCH // L):
        v = idx_v[pl.ds(c * L, L)]
        jl = iota + c * L
        valid = ((v >= seg_lo) & (v < seg_lo + ROWS_PER_TILE) & (v != 0))
        addr = jnp.clip(v - seg_lo, 0, ROWS_PER_TILE - 1)
        for lane in range(L):
            plsc.store_scatter(winner, [addr], jl,
                               mask=valid & (iota == lane))

    cnt = jnp.int32(0)
    for c in range(IDX_PER_BATCH // L):
        v = idx_v[pl.ds(c * L, L)]
        jl = iota + c * L
        valid0 = (v >= seg_lo) & (v < seg_lo + ROWS_PER_TILE) & (v != 0)
        addr = jnp.clip(v - seg_lo, 0, ROWS_PER_TILE - 1)
        w = plsc.load_gather(winner, [addr], mask=valid0)
        keep = valid0 & (w == jl)
        mi = keep.astype(jnp.int32)
        incl = plsc.cumsum(mi)
        pos = cnt + incl - mi
        plsc.store_scatter(list_t, [pos // CHUNK, pos % CHUNK],
                           b * 4096 + v, mask=keep)
        plsc.store_scatter(list_j, [pos // CHUNK, pos % CHUNK],
                           b * IDX_PER_BATCH + jl, mask=keep)
        cnt = cnt + jnp.sum(mi)

    n = cnt
    ceil = ((n + CHUNK - 1) // CHUNK) * CHUNK
    last_i = jnp.maximum(n - 1, 0)
    lt = plsc.load_gather(
        list_t, [jnp.full((L,), last_i // CHUNK, jnp.int32),
                 jnp.full((L,), last_i % CHUNK, jnp.int32)])
    lj = plsc.load_gather(
        list_j, [jnp.full((L,), last_i // CHUNK, jnp.int32),
                 jnp.full((L,), last_i % CHUNK, jnp.int32)])
    for k in range(CHUNK // L):
        pos2 = n + k * L + iota
        m = pos2 < ceil
        pc = jnp.clip(pos2, 0, IDX_PER_BATCH - 1)
        plsc.store_scatter(list_t, [pc // CHUNK, pc % CHUNK], lt, mask=m)
        plsc.store_scatter(list_j, [pc // CHUNK, pc % CHUNK], lj, mask=m)

    nch = ceil // CHUNK

    def two_chunks(i, carry):
        k = 2 * i

        @pl.when(k < nch)
        def _():
            pltpu.async_copy(img_hbm.at[list_j.at[k]], buf_a, semg0).wait()
            pltpu.async_copy(buf_a, out_hbm.at[list_t.at[k]], semp0).wait()

        @pl.when(k + 1 < nch)
        def _():
            pltpu.async_copy(img_hbm.at[list_j.at[k + 1]], buf_b, semg1).wait()
            pltpu.async_copy(buf_b, out_hbm.at[list_t.at[k + 1]], semp1).wait()

        return carry

    lax.fori_loop(0, (nch + 1) // 2, two_chunks, jnp.int32(0))


@jax.jit
def _interleave(img_flat, text_flat, vi_flat):
    nrows, d = text_flat.shape
    br = 512
    copied = pl.pallas_call(
        _copy_body,
        grid=(nrows // br,),
        in_specs=[pl.BlockSpec((br, d), lambda i: (i, 0))],
        out_specs=pl.BlockSpec((br, d), lambda i: (i, 0)),
        out_shape=jax.ShapeDtypeStruct((nrows, d), text_flat.dtype),
    )(text_flat)

    mesh = plsc.VectorSubcoreMesh(core_axis_name="c", subcore_axis_name="s")
    kern = pl_mpmd._mpmd_map(
        [(mesh, _sc_body)],
        jax.ShapeDtypeStruct((nrows, d), text_flat.dtype),
        input_output_aliases={1: 0},
        scratch_types=[
            pltpu.VMEM((IDX_PER_BATCH,), jnp.int32),
            pltpu.VMEM((ROWS_PER_TILE,), jnp.int32),
            pltpu.VMEM((NCH, CHUNK), jnp.int32),
            pltpu.VMEM((NCH, CHUNK), jnp.int32),
            pltpu.VMEM((CHUNK, 2048), jnp.float32),
            pltpu.VMEM((CHUNK, 2048), jnp.float32),
            pltpu.SemaphoreType.DMA,
            pltpu.SemaphoreType.DMA,
            pltpu.SemaphoreType.DMA,
            pltpu.SemaphoreType.DMA,
        ],
        compiler_params=pltpu.CompilerParams(needs_layout_passes=False),
    )
    return kern(img_flat, copied, vi_flat)


def kernel(image_embeddings, text_embeddings, vision_indices):
    B, S, D = text_embeddings.shape
    img_flat = image_embeddings.reshape(-1, D)
    text_flat = text_embeddings.reshape(B * S, D)
    vi_flat = vision_indices.astype(jnp.int32).reshape(-1)
    out = _interleave(img_flat, text_flat, vi_flat)
    return out.reshape(B, S, D)
